# baseline (speedup 1.0000x reference)
import functools

import jax
import jax.numpy as jnp
from jax.experimental import pallas as pl
from jax.experimental.pallas import tpu as pltpu

_BV = 1024
_BR = 8


def _rotl(x, d):
    return jnp.left_shift(x, jnp.uint32(d)) | jnp.right_shift(x, jnp.uint32(32 - d))


def _threefry_bits(k0, k1, cols_u32, shape):
    ks2 = k0 ^ k1 ^ jnp.uint32(0x1BD11BDA)
    x0 = jnp.broadcast_to(k0, shape)
    x1 = jnp.broadcast_to(cols_u32 + k1, shape)
    rots = ((13, 15, 26, 6), (17, 29, 16, 24))
    ksv = (k0, k1, ks2)
    for r in range(5):
        for d in rots[r % 2]:
            x0 = x0 + x1
            x1 = _rotl(x1, d)
            x1 = x1 ^ x0
        x0 = x0 + ksv[(r + 1) % 3]
        x1 = x1 + ksv[(r + 2) % 3] + jnp.uint32(r + 1)
    return x0 ^ x1


def _body(logits_ref, k0_ref, k1_ref, st_ref, nz_ref, out_ref, bv_ref, bi_ref,
          *, nv, vocab):
    v = pl.program_id(1)

    @pl.when(v == 0)
    def _():
        bv_ref[...] = jnp.full((_BR, 1), -jnp.inf, jnp.float32)
        bi_ref[...] = jnp.full((_BR, 1), jnp.int32(2147483647), jnp.int32)

    cols = jax.lax.broadcasted_iota(jnp.int32, (1, _BV), 1) + v * _BV
    k0 = k0_ref[...]
    k1 = k1_ref[...]
    bits = _threefry_bits(k0, k1, cols.astype(jnp.uint32), (_BR, _BV))

    mant = jnp.right_shift(bits, jnp.uint32(9)) | jnp.uint32(0x3F800000)
    u = jax.lax.bitcast_convert_type(mant, jnp.float32) - jnp.float32(1.0)
    g = -jnp.log(u + jnp.float32(1e-20))
    noise = -jnp.log(g + jnp.float32(1e-20))

    scaled = logits_ref[...] / st_ref[...]
    pert = scaled + jnp.where(nz_ref[...] != 0, noise, jnp.float32(0.0))
    pert = jnp.where(cols < vocab, pert, -jnp.inf)

    m = jnp.max(pert, axis=1, keepdims=True)
    idx = jnp.min(jnp.where(pert == m, cols, jnp.int32(2147483647)),
                  axis=1, keepdims=True)

    bv = bv_ref[...]
    bi = bi_ref[...]
    take = (m > bv) | ((m == bv) & (idx < bi))
    bv_ref[...] = jnp.where(take, m, bv)
    bi_ref[...] = jnp.where(take, idx, bi)

    @pl.when(v == nv - 1)
    def _():
        out_ref[...] = bi_ref[...]


def kernel(logits, temperature, seed, pos, apply_temperature):
    rows, vocab = logits.shape
    logits = logits.astype(jnp.float32)

    kd = jax.vmap(
        lambda s, p: jax.random.key_data(jax.random.fold_in(jax.random.key(s), p))
    )(seed, pos)
    k0 = kd[:, 0:1]
    k1 = kd[:, 1:2]

    at = jnp.asarray(apply_temperature)
    safe_t = jnp.where(temperature == 0.0, jnp.float32(1.0), temperature)
    st_eff = jnp.where(at != 0, safe_t, jnp.float32(1.0))[:, None]
    nz = (temperature != 0.0).astype(jnp.int32)[:, None]

    nv = pl.cdiv(vocab, _BV)
    nr = rows // _BR
    out = pl.pallas_call(
        functools.partial(_body, nv=nv, vocab=vocab),
        grid=(nr, nv),
        in_specs=[
            pl.BlockSpec((_BR, _BV), lambda r, v: (r, v)),
            pl.BlockSpec((_BR, 1), lambda r, v: (r, 0)),
            pl.BlockSpec((_BR, 1), lambda r, v: (r, 0)),
            pl.BlockSpec((_BR, 1), lambda r, v: (r, 0)),
            pl.BlockSpec((_BR, 1), lambda r, v: (r, 0)),
        ],
        out_specs=pl.BlockSpec((_BR, 1), lambda r, v: (r, 0)),
        out_shape=jax.ShapeDtypeStruct((rows, 1), jnp.int32),
        scratch_shapes=[
            pltpu.VMEM((_BR, 1), jnp.float32),
            pltpu.VMEM((_BR, 1), jnp.int32),
        ],
    )(logits, k0, k1, st_eff, nz)
    return out[:, 0]


# speedup vs baseline: 3.6837x; 3.6837x over previous
import functools

import jax
import jax.numpy as jnp
from jax.experimental import pallas as pl
from jax.experimental.pallas import tpu as pltpu

_BV = 2048
_CV = 1024
_CR = 8


def _rotl(x, d):
    return jnp.left_shift(x, jnp.uint32(d)) | jnp.right_shift(x, jnp.uint32(32 - d))


def _threefry_bits(k0, k1, x1_init, shape):
    ks2 = k0 ^ k1 ^ jnp.uint32(0x1BD11BDA)
    x0 = jnp.broadcast_to(k0, shape)
    x1 = jnp.broadcast_to(x1_init, shape)
    rots = ((13, 15, 26, 6), (17, 29, 16, 24))
    ksv = (k0, k1, ks2)
    for r in range(5):
        for d in rots[r % 2]:
            x0 = x0 + x1
            x1 = _rotl(x1, d)
            x1 = x1 ^ x0
        x0 = x0 + ksv[(r + 1) % 3]
        x1 = x1 + ksv[(r + 2) % 3] + jnp.uint32(r + 1)
    return x0 ^ x1


def _body(logits_ref, k0_ref, k1_ref, st_ref, nz_ref, out_ref, bv_ref, bi_ref,
          *, nv, vocab, rows):
    v = pl.program_id(0)

    @pl.when(v == 0)
    def _():
        bv_ref[...] = jnp.full((rows, _BV), -jnp.inf, jnp.float32)
        bi_ref[...] = jnp.full((rows, _BV), jnp.int32(2147483647), jnp.int32)

    for r in range(rows // _CR):
        rs = pl.ds(r * _CR, _CR)
        k0 = k0_ref[rs, :]
        k1 = k1_ref[rs, :]
        st = st_ref[rs, :]
        nz = nz_ref[rs, :]
        for c in range(_BV // _CV):
            cols = (jax.lax.broadcasted_iota(jnp.int32, (1, _CV), 1)
                    + (v * _BV + c * _CV))
            bits = _threefry_bits(k0, k1, cols.astype(jnp.uint32) + k1,
                                  (_CR, _CV))
            mant = jnp.right_shift(bits, jnp.uint32(9)) | jnp.uint32(0x3F800000)
            u = jax.lax.bitcast_convert_type(mant, jnp.float32) - jnp.float32(1.0)
            g = -jnp.log(u + jnp.float32(1e-20))
            noise = -jnp.log(g + jnp.float32(1e-20))

            scaled = logits_ref[rs, pl.ds(c * _CV, _CV)] / st
            pert = scaled + jnp.where(nz != 0, noise, jnp.float32(0.0))
            pert = jnp.where(cols < vocab, pert, -jnp.inf)

            cs = pl.ds(c * _CV, _CV)
            bv = bv_ref[rs, cs]
            take = pert > bv
            bv_ref[rs, cs] = jnp.where(take, pert, bv)
            bi_ref[rs, cs] = jnp.where(take, jnp.broadcast_to(cols, (_CR, _CV)),
                                       bi_ref[rs, cs])

    @pl.when(v == nv - 1)
    def _():
        bv = bv_ref[...]
        m = jnp.max(bv, axis=1, keepdims=True)
        idx = jnp.min(jnp.where(bv == m, bi_ref[...], jnp.int32(2147483647)),
                      axis=1, keepdims=True)
        out_ref[...] = idx


def kernel(logits, temperature, seed, pos, apply_temperature):
    rows, vocab = logits.shape
    logits = logits.astype(jnp.float32)

    kd = jax.vmap(
        lambda s, p: jax.random.key_data(jax.random.fold_in(jax.random.key(s), p))
    )(seed, pos)
    k0 = kd[:, 0:1]
    k1 = kd[:, 1:2]

    at = jnp.asarray(apply_temperature)
    safe_t = jnp.where(temperature == 0.0, jnp.float32(1.0), temperature)
    st_eff = jnp.where(at != 0, safe_t, jnp.float32(1.0))[:, None]
    nz = (temperature != 0.0).astype(jnp.int32)[:, None]

    nv = pl.cdiv(vocab, _BV)
    out = pl.pallas_call(
        functools.partial(_body, nv=nv, vocab=vocab, rows=rows),
        grid=(nv,),
        in_specs=[
            pl.BlockSpec((rows, _BV), lambda v: (0, v)),
            pl.BlockSpec((rows, 1), lambda v: (0, 0)),
            pl.BlockSpec((rows, 1), lambda v: (0, 0)),
            pl.BlockSpec((rows, 1), lambda v: (0, 0)),
            pl.BlockSpec((rows, 1), lambda v: (0, 0)),
        ],
        out_specs=pl.BlockSpec((rows, 1), lambda v: (0, 0)),
        out_shape=jax.ShapeDtypeStruct((rows, 1), jnp.int32),
        scratch_shapes=[
            pltpu.VMEM((rows, _BV), jnp.float32),
            pltpu.VMEM((rows, _BV), jnp.int32),
        ],
    )(logits, k0, k1, st_eff, nz)
    return out[:, 0]


# speedup vs baseline: 3.8004x; 1.0317x over previous
import functools

import jax
import jax.numpy as jnp
from jax.experimental import pallas as pl
from jax.experimental.pallas import tpu as pltpu

_BV = 2048
_CV = 1024
_CR = 8


def _rotl(x, d):
    return jnp.left_shift(x, jnp.uint32(d)) | jnp.right_shift(x, jnp.uint32(32 - d))


def _threefry_bits(k0, k1, x1_init, shape):
    ks2 = k0 ^ k1 ^ jnp.uint32(0x1BD11BDA)
    x0 = jnp.broadcast_to(k0, shape)
    x1 = jnp.broadcast_to(x1_init, shape)
    rots = ((13, 15, 26, 6), (17, 29, 16, 24))
    ksv = (k0, k1, ks2)
    inj1 = tuple(ksv[(r + 1) % 3] for r in range(5))
    inj2 = tuple(ksv[(r + 2) % 3] + jnp.uint32(r + 1) for r in range(5))
    for r in range(5):
        for d in rots[r % 2]:
            x0 = x0 + x1
            x1 = _rotl(x1, d)
            x1 = x1 ^ x0
        x0 = x0 + inj1[r]
        x1 = x1 + inj2[r]
    return x0 ^ x1


def _body(logits_ref, k0_ref, k1_ref, st_ref, nz_ref, out_ref, bv_ref, bi_ref,
          *, nv, vocab, rows):
    v = pl.program_id(0)

    @pl.when(v == 0)
    def _():
        bv_ref[...] = jnp.full((rows, _BV), -jnp.inf, jnp.float32)
        bi_ref[...] = jnp.full((rows, _BV), jnp.int32(2147483647), jnp.int32)

    for r in range(rows // _CR):
        rs = pl.ds(r * _CR, _CR)
        k0 = k0_ref[rs, :]
        k1 = k1_ref[rs, :]
        st = st_ref[rs, :]
        nz = nz_ref[rs, :]
        for c in range(_BV // _CV):
            cols = (jax.lax.broadcasted_iota(jnp.int32, (1, _CV), 1)
                    + (v * _BV + c * _CV))
            bits = _threefry_bits(k0, k1, cols.astype(jnp.uint32) + k1,
                                  (_CR, _CV))
            mant = jnp.right_shift(bits, jnp.uint32(9)) | jnp.uint32(0x3F800000)
            u = jax.lax.bitcast_convert_type(mant, jnp.float32) - jnp.float32(1.0)
            g = -jnp.log(u + jnp.float32(1e-20))
            noise = -jnp.log(g + jnp.float32(1e-20))

            scaled = logits_ref[rs, pl.ds(c * _CV, _CV)] / st
            pert = scaled + noise * nz
            pert = jnp.where(cols < vocab, pert, -jnp.inf)

            cs = pl.ds(c * _CV, _CV)
            bv = bv_ref[rs, cs]
            take = pert > bv
            bv_ref[rs, cs] = jnp.where(take, pert, bv)
            bi_ref[rs, cs] = jnp.where(take, jnp.broadcast_to(cols, (_CR, _CV)),
                                       bi_ref[rs, cs])

    @pl.when(v == nv - 1)
    def _():
        bv = bv_ref[...]
        m = jnp.max(bv, axis=1, keepdims=True)
        idx = jnp.min(jnp.where(bv == m, bi_ref[...], jnp.int32(2147483647)),
                      axis=1, keepdims=True)
        out_ref[...] = idx


def kernel(logits, temperature, seed, pos, apply_temperature):
    rows, vocab = logits.shape
    logits = logits.astype(jnp.float32)

    kd = jax.vmap(
        lambda s, p: jax.random.key_data(jax.random.fold_in(jax.random.key(s), p))
    )(seed, pos)
    k0 = kd[:, 0:1]
    k1 = kd[:, 1:2]

    at = jnp.asarray(apply_temperature)
    safe_t = jnp.where(temperature == 0.0, jnp.float32(1.0), temperature)
    st_eff = jnp.where(at != 0, safe_t, jnp.float32(1.0))[:, None]
    nz = (temperature != 0.0).astype(jnp.float32)[:, None]

    nv = pl.cdiv(vocab, _BV)
    out = pl.pallas_call(
        functools.partial(_body, nv=nv, vocab=vocab, rows=rows),
        grid=(nv,),
        in_specs=[
            pl.BlockSpec((rows, _BV), lambda v: (0, v)),
            pl.BlockSpec((rows, 1), lambda v: (0, 0)),
            pl.BlockSpec((rows, 1), lambda v: (0, 0)),
            pl.BlockSpec((rows, 1), lambda v: (0, 0)),
            pl.BlockSpec((rows, 1), lambda v: (0, 0)),
        ],
        out_specs=pl.BlockSpec((rows, 1), lambda v: (0, 0)),
        out_shape=jax.ShapeDtypeStruct((rows, 1), jnp.int32),
        scratch_shapes=[
            pltpu.VMEM((rows, _BV), jnp.float32),
            pltpu.VMEM((rows, _BV), jnp.int32),
        ],
    )(logits, k0, k1, st_eff, nz)
    return out[:, 0]
